# baseline (speedup 1.0000x reference)
import jax
import jax.numpy as jnp
from jax.experimental import pallas as pl

BN = 400


def _add_kernel(x_ref, emb_ref, o_ref):
    emb = emb_ref[...]
    t = x_ref.shape[-1] // emb.shape[-1]
    emb_t = jnp.concatenate([emb] * t, axis=-1)
    o_ref[...] = x_ref[...] + emb_t[None, :, :]


def kernel(x, emb_weight):
    batch, n, t, d = x.shape
    x2 = x.reshape(batch, n, t * d)
    out = pl.pallas_call(
        _add_kernel,
        grid=(batch, n // BN),
        in_specs=[
            pl.BlockSpec((1, BN, t * d), lambda b, i: (b, i, 0)),
            pl.BlockSpec((BN, d), lambda b, i: (i, 0)),
        ],
        out_specs=pl.BlockSpec((1, BN, t * d), lambda b, i: (b, i, 0)),
        out_shape=jax.ShapeDtypeStruct((batch, n, t * d), x.dtype),
    )(x2, emb_weight)
    return out.reshape(batch, n, t, d)


# speedup vs baseline: 1.0062x; 1.0062x over previous
import jax
import jax.numpy as jnp
from jax.experimental import pallas as pl

BN = 1000


def _add_kernel(x_ref, emb_ref, o_ref):
    emb = emb_ref[...]
    d = emb.shape[-1]
    t = x_ref.shape[-1] // d
    for i in range(t):
        sl = slice(i * d, (i + 1) * d)
        o_ref[0, :, sl] = x_ref[0, :, sl] + emb


def kernel(x, emb_weight):
    batch, n, t, d = x.shape
    x2 = x.reshape(batch, n, t * d)
    out = pl.pallas_call(
        _add_kernel,
        grid=(batch, n // BN),
        in_specs=[
            pl.BlockSpec((1, BN, t * d), lambda b, i: (b, i, 0)),
            pl.BlockSpec((BN, d), lambda b, i: (i, 0)),
        ],
        out_specs=pl.BlockSpec((1, BN, t * d), lambda b, i: (b, i, 0)),
        out_shape=jax.ShapeDtypeStruct((batch, n, t * d), x.dtype),
    )(x2, emb_weight)
    return out.reshape(batch, n, t, d)


# speedup vs baseline: 1.6622x; 1.6520x over previous
import jax
import jax.numpy as jnp
from jax.experimental import pallas as pl

BN = 1000


def _add_kernel(x_ref, emb_ref, o_ref):
    o_ref[...] = x_ref[...] + emb_ref[...][None, :, None, :]


def kernel(x, emb_weight):
    batch, n, t, d = x.shape
    return pl.pallas_call(
        _add_kernel,
        grid=(batch, n // BN),
        in_specs=[
            pl.BlockSpec((1, BN, t, d), lambda b, i: (b, i, 0, 0)),
            pl.BlockSpec((BN, d), lambda b, i: (i, 0)),
        ],
        out_specs=pl.BlockSpec((1, BN, t, d), lambda b, i: (b, i, 0, 0)),
        out_shape=jax.ShapeDtypeStruct((batch, n, t, d), x.dtype),
    )(x, emb_weight)
